# baseline (speedup 1.0000x reference)
import jax
import jax.numpy as jnp
import numpy as np
from jax.experimental import pallas as pl

_MININT = np.int32(-2147483648)
_MAXPOS = np.int32(2147483647)


def _dott(x, w):
    return jax.lax.dot_general(x, w, (((1,), (1,)), ((), ())),
                               preferred_element_type=jnp.float32)


def _kwta(x, key_src, k, tri_bf16):
    skey = jax.lax.bitcast_convert_type(key_src + 0.0, jnp.int32)
    skey = jnp.where(skey < 0, skey ^ _MAXPOS, skey)

    skey_t = skey.T
    k_row = k.T

    def body(i, t_u):
        bit = jax.lax.shift_left(jnp.int32(1), jnp.int32(31) - i)
        cand = t_u | bit
        thr = cand ^ _MININT
        cnt = jnp.sum((skey_t >= thr).astype(jnp.int32), axis=0, keepdims=True)
        return jnp.where(cnt >= k_row, cand, t_u)

    t_u = jax.lax.fori_loop(0, 32, body, jnp.zeros_like(k_row), unroll=4)
    t_s = (t_u ^ _MININT).T

    gt = skey > t_s
    c_gt = jnp.sum(gt.astype(jnp.int32), axis=1, keepdims=True)
    eq = skey == t_s
    cum_excl = jnp.dot(eq.astype(jnp.bfloat16), tri_bf16,
                       preferred_element_type=jnp.float32)
    keep = eq & (cum_excl < (k - c_gt).astype(jnp.float32))
    mask = (gt | keep) & (k > 0)
    return jnp.where(mask, x, x / 3.0)


def _body(s_ref, ti_ref, at_ref, wab_ref, wt_ref, b1_ref,
          w12_ref, w22_ref, w32_ref, wl2_ref, wl3_ref, wl4_ref, b2_ref,
          t1_ref, t2_ref, t3_ref, out_ref):
    f32 = jnp.float32
    sa = s_ref[...].astype(jnp.bfloat16)
    tb = ti_ref[...].astype(jnp.bfloat16)
    at = at_ref[...]

    def in_dot(oab, ot, n):
        return (_dott(sa, wab_ref[oab:oab + n, :])
                + _dott(tb, wab_ref[oab + n:oab + 2 * n, :])
                + _dott(at, wt_ref[ot:ot + n, :]))

    h1 = jnp.tanh(in_dot(0, 0, 1024) + b1_ref[:, 0:1024])
    z1 = _dott(h1.astype(jnp.bfloat16), w12_ref[...]) + b2_ref[:, 0:1024]
    k1 = jnp.argmax(z1, axis=1).astype(jnp.int32)[:, None]

    h2 = jnp.tanh(in_dot(2048, 1024, 512) + b1_ref[:, 1024:1536])
    z2 = _dott(h2.astype(jnp.bfloat16), w22_ref[...]) + b2_ref[:, 1024:1536]
    k2 = jnp.argmax(z2, axis=1).astype(jnp.int32)[:, None]

    h3 = jnp.tanh(in_dot(3072, 1536, 128) + b1_ref[:, 1536:1664])
    z3 = _dott(h3.astype(jnp.bfloat16), w32_ref[...]) + b2_ref[:, 1536:1664]
    k3 = jnp.argmax(z3, axis=1).astype(jnp.int32)[:, None]

    x = in_dot(3328, 1664, 1024) + b1_ref[:, 1664:2688]
    x = _kwta(x, x, k1, t1_ref[...])
    x = _dott(x.astype(jnp.bfloat16), wl2_ref[...]) + b2_ref[:, 1664:2176]
    x = _kwta(x, x, k2, t2_ref[...])
    x = _dott(x.astype(jnp.bfloat16), wl3_ref[...]) + b2_ref[:, 2176:2304]
    col = jax.lax.broadcasted_iota(jnp.int32, x.shape, 1)
    key3 = jnp.where(col < 64, x, f32(-1e30))
    x = _kwta(x, key3, k3, t3_ref[...])
    out_ref[...] = (_dott(x.astype(jnp.bfloat16), wl4_ref[...])
                    + b2_ref[:, 2304:2368])


def _tri(n):
    r = jnp.arange(n, dtype=jnp.int32)
    return (r[:, None] < r[None, :]).astype(jnp.bfloat16)


def _wsplit(W, ns):
    wa = W[:, :ns].astype(jnp.bfloat16)
    wb = W[:, ns:2 * ns].astype(jnp.bfloat16)
    wt = jnp.pad(W[:, 2 * ns:], ((0, 0), (0, 128 - (W.shape[1] - 2 * ns)))
                 ).astype(jnp.bfloat16)
    return wa, wb, wt


def kernel(state, task_indicator,
           W_cx1_1, b_cx1_1, W_cx1_2, b_cx1_2,
           W_cx2_1, b_cx2_1, W_cx2_2, b_cx2_2,
           W_cx3_1, b_cx3_1, W_cx3_2, b_cx3_2,
           W_lin1, b_lin1, W_lin2, b_lin2,
           W_lin3, b_lin3, W_lin4, b_lin4):
    B = state.shape[0]
    NS = state.shape[1]
    KM = 2 * NS
    R = 512
    H2, H1, NH = 1024, 512, 64

    at = jnp.pad(task_indicator[:, NS:],
                 ((0, 0), (0, 128 - (task_indicator.shape[1] - NS)))
                 ).astype(jnp.bfloat16)

    w31p = jnp.pad(W_cx3_1, ((0, 64), (0, 0)))
    wab = jnp.concatenate(
        [W_cx1_1[:, :NS], W_cx1_1[:, NS:KM],
         W_cx2_1[:, :NS], W_cx2_1[:, NS:KM],
         w31p[:, :NS], w31p[:, NS:KM],
         W_lin1[:, :NS], W_lin1[:, NS:KM]], axis=0).astype(jnp.bfloat16)
    wt = jnp.concatenate(
        [jnp.pad(W[:, KM:], ((0, 0), (0, 124)))
         for W in (W_cx1_1, W_cx2_1, w31p, W_lin1)], axis=0
    ).astype(jnp.bfloat16)
    b1 = jnp.concatenate(
        [b_cx1_1, b_cx2_1, jnp.pad(b_cx3_1, (0, 64)), b_lin1])[None, :]
    b2 = jnp.concatenate(
        [b_cx1_2, b_cx2_2, jnp.pad(b_cx3_2, (0, 64), constant_values=-1e9),
         b_lin2, jnp.pad(b_lin3, (0, 64)), b_lin4,
         jnp.zeros((64,), jnp.float32)])[None, :]

    w12 = W_cx1_2.astype(jnp.bfloat16)
    w22 = W_cx2_2.astype(jnp.bfloat16)
    w32 = jnp.pad(W_cx3_2, ((0, 64), (0, 64))).astype(jnp.bfloat16)
    wl2 = W_lin2.astype(jnp.bfloat16)
    wl3 = jnp.pad(W_lin3, ((0, 64), (0, 0))).astype(jnp.bfloat16)
    wl4 = jnp.pad(W_lin4, ((0, 0), (0, 64))).astype(jnp.bfloat16)

    t1, t2, t3 = _tri(H2), _tri(H1), _tri(128)

    def const(shape):
        return pl.BlockSpec(shape, lambda i: (0, 0))

    out = pl.pallas_call(
        _body,
        grid=(B // R,),
        in_specs=[
            pl.BlockSpec((R, NS), lambda i: (i, 0)),
            pl.BlockSpec((R, NS), lambda i: (i, 0)),
            pl.BlockSpec((R, 128), lambda i: (i, 0)),
            const(wab.shape), const(wt.shape), const(b1.shape),
            const(w12.shape), const(w22.shape), const(w32.shape),
            const(wl2.shape), const(wl3.shape), const(wl4.shape),
            const(b2.shape),
            const(t1.shape), const(t2.shape), const(t3.shape),
        ],
        out_specs=pl.BlockSpec((R, NH), lambda i: (i, 0)),
        out_shape=jax.ShapeDtypeStruct((B, NH), jnp.float32),
    )(state, task_indicator, at, wab, wt, b1,
      w12, w22, w32, wl2, wl3, wl4, b2, t1, t2, t3)
    return out


# speedup vs baseline: 1.0004x; 1.0004x over previous
import jax
import jax.numpy as jnp
import numpy as np
from jax.experimental import pallas as pl

_MININT = np.int32(-2147483648)
_MAXPOS = np.int32(2147483647)


def _dott(x, w):
    return jax.lax.dot_general(x, w, (((1,), (1,)), ((), ())),
                               preferred_element_type=jnp.float32)


def _kwta(x, key_src, k, tri_bf16):
    skey = jax.lax.bitcast_convert_type(key_src + 0.0, jnp.int32)
    skey = jnp.where(skey < 0, skey ^ _MAXPOS, skey)

    skey_t = skey.T
    k_row = k.T

    def body(i, t_u):
        bit = jax.lax.shift_left(jnp.int32(1), jnp.int32(31) - i)
        cand = t_u | bit
        thr = cand ^ _MININT
        cnt = jnp.sum((skey_t >= thr).astype(jnp.int32), axis=0, keepdims=True)
        return jnp.where(cnt >= k_row, cand, t_u)

    t_u = jax.lax.fori_loop(0, 32, body, jnp.zeros_like(k_row), unroll=4)
    t_s = (t_u ^ _MININT).T

    gt = skey > t_s
    c_gt = jnp.sum(gt.astype(jnp.int32), axis=1, keepdims=True)
    eq = skey == t_s
    cum_excl = jnp.dot(eq.astype(jnp.bfloat16), tri_bf16,
                       preferred_element_type=jnp.float32)
    keep = eq & (cum_excl < (k - c_gt).astype(jnp.float32))
    mask = (gt | keep) & (k > 0)
    return jnp.where(mask, x, x / 3.0)


def _body(s_ref, ti_ref, at_ref, wab_ref, wt_ref, b1_ref,
          w12_ref, w22_ref, w32_ref, wl2_ref, wl3_ref, wl4_ref, b2_ref,
          t1_ref, t2_ref, t3_ref, out_ref):
    f32 = jnp.float32
    sa = s_ref[...].astype(jnp.bfloat16)
    tb = ti_ref[...].astype(jnp.bfloat16)
    at = at_ref[...]

    def in_dot(oab, ot, n):
        return (_dott(sa, wab_ref[oab:oab + n, :])
                + _dott(tb, wab_ref[oab + n:oab + 2 * n, :])
                + _dott(at, wt_ref[ot:ot + n, :]))

    h1 = jnp.tanh(in_dot(0, 0, 1024) + b1_ref[:, 0:1024])
    z1 = _dott(h1.astype(jnp.bfloat16), w12_ref[...]) + b2_ref[:, 0:1024]
    k1 = jnp.argmax(z1, axis=1).astype(jnp.int32)[:, None]

    h2 = jnp.tanh(in_dot(2048, 1024, 512) + b1_ref[:, 1024:1536])
    z2 = _dott(h2.astype(jnp.bfloat16), w22_ref[...]) + b2_ref[:, 1024:1536]
    k2 = jnp.argmax(z2, axis=1).astype(jnp.int32)[:, None]

    h3 = jnp.tanh(in_dot(3072, 1536, 128) + b1_ref[:, 1536:1664])
    z3 = _dott(h3.astype(jnp.bfloat16), w32_ref[...]) + b2_ref[:, 1536:1664]
    k3 = jnp.argmax(z3, axis=1).astype(jnp.int32)[:, None]

    x = in_dot(3328, 1664, 1024) + b1_ref[:, 1664:2688]
    x = _kwta(x, x, k1, t1_ref[...])
    x = _dott(x.astype(jnp.bfloat16), wl2_ref[...]) + b2_ref[:, 1664:2176]
    x = _kwta(x, x, k2, t2_ref[...])
    x = _dott(x.astype(jnp.bfloat16), wl3_ref[...]) + b2_ref[:, 2176:2304]
    col = jax.lax.broadcasted_iota(jnp.int32, x.shape, 1)
    key3 = jnp.where(col < 64, x, f32(-1e30))
    x = _kwta(x, key3, k3, t3_ref[...])
    out_ref[...] = (_dott(x.astype(jnp.bfloat16), wl4_ref[...])
                    + b2_ref[:, 2304:2368])


def _tri(n):
    r = jnp.arange(n, dtype=jnp.int32)
    return (r[:, None] < r[None, :]).astype(jnp.bfloat16)


def kernel(state, task_indicator,
           W_cx1_1, b_cx1_1, W_cx1_2, b_cx1_2,
           W_cx2_1, b_cx2_1, W_cx2_2, b_cx2_2,
           W_cx3_1, b_cx3_1, W_cx3_2, b_cx3_2,
           W_lin1, b_lin1, W_lin2, b_lin2,
           W_lin3, b_lin3, W_lin4, b_lin4):
    B = state.shape[0]
    NS = state.shape[1]
    KM = 2 * NS
    R = 512
    H2, H1, NH = 1024, 512, 64

    at = jnp.pad(task_indicator[:, NS:],
                 ((0, 0), (0, 128 - (task_indicator.shape[1] - NS)))
                 ).astype(jnp.bfloat16)

    w31p = jnp.pad(W_cx3_1, ((0, 64), (0, 0)))
    wab = jnp.concatenate(
        [W_cx1_1[:, :NS], W_cx1_1[:, NS:KM],
         W_cx2_1[:, :NS], W_cx2_1[:, NS:KM],
         w31p[:, :NS], w31p[:, NS:KM],
         W_lin1[:, :NS], W_lin1[:, NS:KM]], axis=0).astype(jnp.bfloat16)
    wt = jnp.concatenate(
        [jnp.pad(W[:, KM:], ((0, 0), (0, 124)))
         for W in (W_cx1_1, W_cx2_1, w31p, W_lin1)], axis=0
    ).astype(jnp.bfloat16)
    b1 = jnp.concatenate(
        [b_cx1_1, b_cx2_1, jnp.pad(b_cx3_1, (0, 64)), b_lin1])[None, :]
    b2 = jnp.concatenate(
        [b_cx1_2, b_cx2_2, jnp.pad(b_cx3_2, (0, 64), constant_values=-1e9),
         b_lin2, jnp.pad(b_lin3, (0, 64)), b_lin4,
         jnp.zeros((64,), jnp.float32)])[None, :]

    w12 = W_cx1_2.astype(jnp.bfloat16)
    w22 = W_cx2_2.astype(jnp.bfloat16)
    w32 = jnp.pad(W_cx3_2, ((0, 64), (0, 64))).astype(jnp.bfloat16)
    wl2 = W_lin2.astype(jnp.bfloat16)
    wl3 = jnp.pad(W_lin3, ((0, 64), (0, 0))).astype(jnp.bfloat16)
    wl4 = jnp.pad(W_lin4, ((0, 0), (0, 64))).astype(jnp.bfloat16)

    t1, t2, t3 = _tri(H2), _tri(H1), _tri(128)

    def const(shape):
        return pl.BlockSpec(shape, lambda i: (0, 0))

    out = pl.pallas_call(
        _body,
        grid=(B // R,),
        in_specs=[
            pl.BlockSpec((R, NS), lambda i: (i, 0)),
            pl.BlockSpec((R, NS), lambda i: (i, 0)),
            pl.BlockSpec((R, 128), lambda i: (i, 0)),
            const(wab.shape), const(wt.shape), const(b1.shape),
            const(w12.shape), const(w22.shape), const(w32.shape),
            const(wl2.shape), const(wl3.shape), const(wl4.shape),
            const(b2.shape),
            const(t1.shape), const(t2.shape), const(t3.shape),
        ],
        out_specs=pl.BlockSpec((R, NH), lambda i: (i, 0)),
        out_shape=jax.ShapeDtypeStruct((B, NH), jnp.float32),
    )(state, task_indicator, at, wab, wt, b1,
      w12, w22, w32, wl2, wl3, wl4, b2, t1, t2, t3)
    return out
